# baseline (speedup 1.0000x reference)
import functools

import jax
import jax.numpy as jnp
from jax import lax
from jax.experimental import pallas as pl
from jax.experimental.pallas import tpu as pltpu
from jax.experimental.pallas import tpu_sc as plsc

_N_PAD = 10240
_WORDS = 384
_ROW_BLK = 512
_TGT_BLK = 256


def _h_body(x_ref, w1_ref, b1_ref, w2_ref, b2_ref, out_ref):
    xb = x_ref[...]
    t = jnp.maximum(jnp.dot(xb, w1_ref[...], preferred_element_type=jnp.float32)
                    + b1_ref[...], 0.0)
    t = jnp.maximum(jnp.dot(t, w2_ref[...], preferred_element_type=jnp.float32)
                    + b2_ref[...], 0.0)
    hb = xb + t
    out_ref[...] = hb.astype(jnp.bfloat16).reshape(_ROW_BLK // 32, 32, 128)


def _main_body(r0_ref, r1_ref, xi_ref, xj_ref, h3_ref,
               wcn1_ref, bcn1_ref, wcn2_ref, bcn2_ref,
               wij_ref, bij_ref, wl1_ref, bl1_ref, wl2_ref, bl2_ref,
               beta_ref, out_ref):
    acc = jnp.zeros((_TGT_BLK, 128), jnp.float32)
    for wt in range(_WORDS // 128):
        sl = slice(wt * 128, (wt + 1) * 128)
        words = r0_ref[:, sl] & r1_ref[:, sl]
        for p in range(32):
            plane = ((words >> p) & 1).astype(jnp.bfloat16)
            hsub = h3_ref[sl, p, :]
            acc += jnp.dot(plane, hsub, preferred_element_type=jnp.float32)
    xcn2 = jnp.maximum(
        jnp.dot(acc, wcn1_ref[...], preferred_element_type=jnp.float32)
        + bcn1_ref[...], 0.0)
    xcn2 = jnp.maximum(
        jnp.dot(xcn2, wcn2_ref[...], preferred_element_type=jnp.float32)
        + bcn2_ref[...], 0.0)
    xij = jnp.maximum(
        jnp.dot(xi_ref[...] * xj_ref[...], wij_ref[...],
                preferred_element_type=jnp.float32)
        + bij_ref[...], 0.0)
    z = xcn2 * beta_ref[0] + xij
    z = jnp.maximum(
        jnp.dot(z, wl1_ref[...], preferred_element_type=jnp.float32)
        + bl1_ref[...], 0.0)
    out_ref[...] = jnp.dot(z, wl2_ref[...], preferred_element_type=jnp.float32) \
        + bl2_ref[...]


def _sc_gather(abits, x, ti, tj):
    b = ti.shape[0]
    info = plsc.get_sparse_core_info()
    nc, ns = info.num_cores, info.num_subcores
    per_w = b // (nc * ns)
    mesh = plsc.VectorSubcoreMesh(core_axis_name="c", subcore_axis_name="s")

    @functools.partial(
        pl.kernel, mesh=mesh,
        out_type=[
            jax.ShapeDtypeStruct((b, _WORDS), jnp.int32),
            jax.ShapeDtypeStruct((b, _WORDS), jnp.int32),
            jax.ShapeDtypeStruct((b, 128), jnp.float32),
            jax.ShapeDtypeStruct((b, 128), jnp.float32),
        ],
        scratch_types=[
            pltpu.VMEM((per_w,), jnp.int32),
            pltpu.VMEM((per_w, _WORDS), jnp.int32),
            pltpu.VMEM((per_w, 128), jnp.float32),
            pltpu.SemaphoreType.DMA,
        ],
    )
    def k(abits_hbm, x_hbm, ti_hbm, tj_hbm,
          r0_hbm, r1_hbm, xi_hbm, xj_hbm,
          idx_v, rows_v, xrows_v, sem):
        wid = lax.axis_index("s") * nc + lax.axis_index("c")
        base = wid * per_w
        pltpu.sync_copy(ti_hbm.at[pl.ds(base, per_w)], idx_v)
        pltpu.async_copy(abits_hbm.at[idx_v], rows_v, sem).wait()
        pltpu.sync_copy(rows_v, r0_hbm.at[pl.ds(base, per_w)])
        pltpu.async_copy(x_hbm.at[idx_v], xrows_v, sem).wait()
        pltpu.sync_copy(xrows_v, xi_hbm.at[pl.ds(base, per_w)])
        pltpu.sync_copy(tj_hbm.at[pl.ds(base, per_w)], idx_v)
        pltpu.async_copy(abits_hbm.at[idx_v], rows_v, sem).wait()
        pltpu.sync_copy(rows_v, r1_hbm.at[pl.ds(base, per_w)])
        pltpu.async_copy(x_hbm.at[idx_v], xrows_v, sem).wait()
        pltpu.sync_copy(xrows_v, xj_hbm.at[pl.ds(base, per_w)])

    return k(abits, x, ti, tj)


@jax.jit
def kernel(x, edge_index, tar_ei, beta,
           W_xlin1, b_xlin1, W_xlin2, b_xlin2,
           W_cn1, b_cn1, W_cn2, b_cn2,
           W_ij1, b_ij1, W_l1, b_l1, W_l2, b_l2):
    n, d = x.shape
    b = tar_ei.shape[1]

    src = edge_index[0].astype(jnp.int32)
    dst = edge_index[1].astype(jnp.int32)
    keys = (src << 14) | dst
    skeys, = lax.sort([keys], is_stable=False)
    dup = jnp.concatenate(
        [jnp.zeros((1,), jnp.bool_), skeys[1:] == skeys[:-1]])
    su = skeys >> 14
    sv = skeys & 16383
    word_idx = su * _WORDS + (sv >> 5)
    bitval = jnp.where(dup, 0, jnp.left_shift(jnp.int32(1), sv & 31))
    abits = jnp.zeros((n * _WORDS,), jnp.int32).at[word_idx].add(
        bitval, mode="drop").reshape(n, _WORDS)

    ti = tar_ei[0].astype(jnp.int32)
    tj = tar_ei[1].astype(jnp.int32)
    r0, r1, xi, xj = _sc_gather(abits, x, ti, tj)

    x_pad = jnp.pad(x, ((0, _N_PAD - n), (0, 0)))
    h3 = pl.pallas_call(
        _h_body,
        grid=(_N_PAD // _ROW_BLK,),
        in_specs=[
            pl.BlockSpec((_ROW_BLK, 128), lambda i: (i, 0)),
            pl.BlockSpec((128, 128), lambda i: (0, 0)),
            pl.BlockSpec((128,), lambda i: (0,)),
            pl.BlockSpec((128, 128), lambda i: (0, 0)),
            pl.BlockSpec((128,), lambda i: (0,)),
        ],
        out_specs=pl.BlockSpec((_ROW_BLK // 32, 32, 128), lambda i: (i, 0, 0)),
        out_shape=jax.ShapeDtypeStruct((_WORDS, 32, 128), jnp.bfloat16),
    )(x_pad, W_xlin1, b_xlin1, W_xlin2, b_xlin2)
    h3 = jnp.pad(h3[: _N_PAD // 32], ((0, _WORDS - _N_PAD // 32), (0, 0), (0, 0)))

    beta_s = beta.astype(jnp.float32).reshape((1,))

    out = pl.pallas_call(
        _main_body,
        grid=(b // _TGT_BLK,),
        in_specs=[
            pl.BlockSpec((_TGT_BLK, _WORDS), lambda i: (i, 0)),
            pl.BlockSpec((_TGT_BLK, _WORDS), lambda i: (i, 0)),
            pl.BlockSpec((_TGT_BLK, 128), lambda i: (i, 0)),
            pl.BlockSpec((_TGT_BLK, 128), lambda i: (i, 0)),
            pl.BlockSpec((_WORDS, 32, 128), lambda i: (0, 0, 0)),
            pl.BlockSpec((128, 128), lambda i: (0, 0)),
            pl.BlockSpec((128,), lambda i: (0,)),
            pl.BlockSpec((128, 128), lambda i: (0, 0)),
            pl.BlockSpec((128,), lambda i: (0,)),
            pl.BlockSpec((128, 128), lambda i: (0, 0)),
            pl.BlockSpec((128,), lambda i: (0,)),
            pl.BlockSpec((128, 128), lambda i: (0, 0)),
            pl.BlockSpec((128,), lambda i: (0,)),
            pl.BlockSpec((128, 1), lambda i: (0, 0)),
            pl.BlockSpec((1,), lambda i: (0,)),
            pl.BlockSpec(memory_space=pltpu.SMEM),
        ],
        out_specs=pl.BlockSpec((_TGT_BLK, 1), lambda i: (i, 0)),
        out_shape=jax.ShapeDtypeStruct((b, 1), jnp.float32),
    )(r0, r1, xi, xj, h3,
      W_cn1, b_cn1, W_cn2, b_cn2,
      W_ij1, b_ij1, W_l1, b_l1, W_l2, b_l2,
      beta_s)
    return out


# speedup vs baseline: 1.0707x; 1.0707x over previous
import functools

import jax
import jax.numpy as jnp
from jax import lax
from jax.experimental import pallas as pl
from jax.experimental.pallas import tpu as pltpu
from jax.experimental.pallas import tpu_sc as plsc

_N_PAD = 10240
_WORDS = 384
_ROW_BLK = 512
_TGT_BLK = 256


def _h_body(x_ref, w1_ref, b1_ref, w2_ref, b2_ref, out_ref):
    xb = x_ref[...]
    t = jnp.maximum(jnp.dot(xb, w1_ref[...], preferred_element_type=jnp.float32)
                    + b1_ref[...], 0.0)
    t = jnp.maximum(jnp.dot(t, w2_ref[...], preferred_element_type=jnp.float32)
                    + b2_ref[...], 0.0)
    hb = xb + t
    out_ref[...] = hb.reshape(_ROW_BLK // 32, 32, 128)


def _main_body(r0_ref, r1_ref, xi_ref, xj_ref, h3_ref,
               wcn1_ref, bcn1_ref, wcn2_ref, bcn2_ref,
               wij_ref, bij_ref, wl1_ref, bl1_ref, wl2_ref, bl2_ref,
               beta_ref, out_ref):
    acc = jnp.zeros((_TGT_BLK, 128), jnp.float32)
    for wt in range(_WORDS // 128):
        sl = slice(wt * 128, (wt + 1) * 128)
        words = r0_ref[:, sl] & r1_ref[:, sl]
        for p in range(32):
            plane = ((words >> p) & 1).astype(jnp.float32)
            hsub = h3_ref[sl, p, :]
            acc += jnp.dot(plane, hsub, preferred_element_type=jnp.float32)
    xcn2 = jnp.maximum(
        jnp.dot(acc, wcn1_ref[...], preferred_element_type=jnp.float32)
        + bcn1_ref[...], 0.0)
    xcn2 = jnp.maximum(
        jnp.dot(xcn2, wcn2_ref[...], preferred_element_type=jnp.float32)
        + bcn2_ref[...], 0.0)
    xij = jnp.maximum(
        jnp.dot(xi_ref[...] * xj_ref[...], wij_ref[...],
                preferred_element_type=jnp.float32)
        + bij_ref[...], 0.0)
    z = xcn2 * beta_ref[0] + xij
    z = jnp.maximum(
        jnp.dot(z, wl1_ref[...], preferred_element_type=jnp.float32)
        + bl1_ref[...], 0.0)
    out_ref[...] = jnp.dot(z, wl2_ref[...], preferred_element_type=jnp.float32) \
        + bl2_ref[...]


def _sc_gather(abits, x, ti, tj):
    b = ti.shape[0]
    info = plsc.get_sparse_core_info()
    nc, ns = info.num_cores, info.num_subcores
    per_w = b // (nc * ns)
    mesh = plsc.VectorSubcoreMesh(core_axis_name="c", subcore_axis_name="s")

    @functools.partial(
        pl.kernel, mesh=mesh,
        out_type=[
            jax.ShapeDtypeStruct((b, _WORDS), jnp.int32),
            jax.ShapeDtypeStruct((b, _WORDS), jnp.int32),
            jax.ShapeDtypeStruct((b, 128), jnp.float32),
            jax.ShapeDtypeStruct((b, 128), jnp.float32),
        ],
        scratch_types=[
            pltpu.VMEM((per_w,), jnp.int32),
            pltpu.VMEM((per_w, _WORDS), jnp.int32),
            pltpu.VMEM((per_w, 128), jnp.float32),
            pltpu.SemaphoreType.DMA,
        ],
    )
    def k(abits_hbm, x_hbm, ti_hbm, tj_hbm,
          r0_hbm, r1_hbm, xi_hbm, xj_hbm,
          idx_v, rows_v, xrows_v, sem):
        wid = lax.axis_index("s") * nc + lax.axis_index("c")
        base = wid * per_w
        pltpu.sync_copy(ti_hbm.at[pl.ds(base, per_w)], idx_v)
        pltpu.async_copy(abits_hbm.at[idx_v], rows_v, sem).wait()
        pltpu.sync_copy(rows_v, r0_hbm.at[pl.ds(base, per_w)])
        pltpu.async_copy(x_hbm.at[idx_v], xrows_v, sem).wait()
        pltpu.sync_copy(xrows_v, xi_hbm.at[pl.ds(base, per_w)])
        pltpu.sync_copy(tj_hbm.at[pl.ds(base, per_w)], idx_v)
        pltpu.async_copy(abits_hbm.at[idx_v], rows_v, sem).wait()
        pltpu.sync_copy(rows_v, r1_hbm.at[pl.ds(base, per_w)])
        pltpu.async_copy(x_hbm.at[idx_v], xrows_v, sem).wait()
        pltpu.sync_copy(xrows_v, xj_hbm.at[pl.ds(base, per_w)])

    return k(abits, x, ti, tj)


@jax.jit
def kernel(x, edge_index, tar_ei, beta,
           W_xlin1, b_xlin1, W_xlin2, b_xlin2,
           W_cn1, b_cn1, W_cn2, b_cn2,
           W_ij1, b_ij1, W_l1, b_l1, W_l2, b_l2):
    n, d = x.shape
    b = tar_ei.shape[1]

    src = edge_index[0].astype(jnp.int32)
    dst = edge_index[1].astype(jnp.int32)
    keys = (src << 14) | dst
    skeys, = lax.sort([keys], is_stable=False)
    dup = jnp.concatenate(
        [jnp.zeros((1,), jnp.bool_), skeys[1:] == skeys[:-1]])
    su = skeys >> 14
    sv = skeys & 16383
    word_idx = su * _WORDS + (sv >> 5)
    bitval = jnp.where(dup, 0, jnp.left_shift(jnp.int32(1), sv & 31))
    abits = jnp.zeros((n * _WORDS,), jnp.int32).at[word_idx].add(
        bitval, mode="drop").reshape(n, _WORDS)

    ti = tar_ei[0].astype(jnp.int32)
    tj = tar_ei[1].astype(jnp.int32)
    r0, r1, xi, xj = _sc_gather(abits, x, ti, tj)

    x_pad = jnp.pad(x, ((0, _N_PAD - n), (0, 0)))
    h3 = pl.pallas_call(
        _h_body,
        grid=(_N_PAD // _ROW_BLK,),
        in_specs=[
            pl.BlockSpec((_ROW_BLK, 128), lambda i: (i, 0)),
            pl.BlockSpec((128, 128), lambda i: (0, 0)),
            pl.BlockSpec((128,), lambda i: (0,)),
            pl.BlockSpec((128, 128), lambda i: (0, 0)),
            pl.BlockSpec((128,), lambda i: (0,)),
        ],
        out_specs=pl.BlockSpec((_ROW_BLK // 32, 32, 128), lambda i: (i, 0, 0)),
        out_shape=jax.ShapeDtypeStruct((_WORDS, 32, 128), jnp.float32),
    )(x_pad, W_xlin1, b_xlin1, W_xlin2, b_xlin2)
    h3 = jnp.pad(h3[: _N_PAD // 32], ((0, _WORDS - _N_PAD // 32), (0, 0), (0, 0)))

    beta_s = beta.astype(jnp.float32).reshape((1,))

    out = pl.pallas_call(
        _main_body,
        grid=(b // _TGT_BLK,),
        in_specs=[
            pl.BlockSpec((_TGT_BLK, _WORDS), lambda i: (i, 0)),
            pl.BlockSpec((_TGT_BLK, _WORDS), lambda i: (i, 0)),
            pl.BlockSpec((_TGT_BLK, 128), lambda i: (i, 0)),
            pl.BlockSpec((_TGT_BLK, 128), lambda i: (i, 0)),
            pl.BlockSpec((_WORDS, 32, 128), lambda i: (0, 0, 0)),
            pl.BlockSpec((128, 128), lambda i: (0, 0)),
            pl.BlockSpec((128,), lambda i: (0,)),
            pl.BlockSpec((128, 128), lambda i: (0, 0)),
            pl.BlockSpec((128,), lambda i: (0,)),
            pl.BlockSpec((128, 128), lambda i: (0, 0)),
            pl.BlockSpec((128,), lambda i: (0,)),
            pl.BlockSpec((128, 128), lambda i: (0, 0)),
            pl.BlockSpec((128,), lambda i: (0,)),
            pl.BlockSpec((128, 1), lambda i: (0, 0)),
            pl.BlockSpec((1,), lambda i: (0,)),
            pl.BlockSpec(memory_space=pltpu.SMEM),
        ],
        out_specs=pl.BlockSpec((_TGT_BLK, 1), lambda i: (i, 0)),
        out_shape=jax.ShapeDtypeStruct((b, 1), jnp.float32),
    )(r0, r1, xi, xj, h3,
      W_cn1, b_cn1, W_cn2, b_cn2,
      W_ij1, b_ij1, W_l1, b_l1, W_l2, b_l2,
      beta_s)
    return out


# speedup vs baseline: 1.2123x; 1.1323x over previous
import functools

import jax
import jax.numpy as jnp
from jax import lax
from jax.experimental import pallas as pl
from jax.experimental.pallas import tpu as pltpu
from jax.experimental.pallas import tpu_sc as plsc

_N_PAD = 10240
_WORDS = 384
_ROW_BLK = 512
_TGT_BLK = 512


def _h_body(x_ref, w1_ref, b1_ref, w2_ref, b2_ref, out_ref):
    xb = x_ref[...]
    t = jnp.maximum(jnp.dot(xb, w1_ref[...], preferred_element_type=jnp.float32)
                    + b1_ref[...], 0.0)
    t = jnp.maximum(jnp.dot(t, w2_ref[...], preferred_element_type=jnp.float32)
                    + b2_ref[...], 0.0)
    hb = xb + t
    out_ref[...] = hb.reshape(_ROW_BLK // 32, 32, 128)


def _main_body(r0_ref, r1_ref, xi_ref, xj_ref, h3_ref,
               wcn1_ref, bcn1_ref, wcn2_ref, bcn2_ref,
               wij_ref, bij_ref, wl1_ref, bl1_ref, wl2_ref, bl2_ref,
               beta_ref, out_ref):
    acc = jnp.zeros((_TGT_BLK, 128), jnp.float32)
    for wt in range(_WORDS // 128):
        sl = slice(wt * 128, (wt + 1) * 128)
        words = r0_ref[:, sl] & r1_ref[:, sl]
        for p in range(32):
            plane = ((words >> p) & 1).astype(jnp.float32)
            hsub = h3_ref[sl, p, :]
            acc += jnp.dot(plane, hsub, preferred_element_type=jnp.float32)
    xcn2 = jnp.maximum(
        jnp.dot(acc, wcn1_ref[...], preferred_element_type=jnp.float32)
        + bcn1_ref[...], 0.0)
    xcn2 = jnp.maximum(
        jnp.dot(xcn2, wcn2_ref[...], preferred_element_type=jnp.float32)
        + bcn2_ref[...], 0.0)
    xij = jnp.maximum(
        jnp.dot(xi_ref[...] * xj_ref[...], wij_ref[...],
                preferred_element_type=jnp.float32)
        + bij_ref[...], 0.0)
    z = xcn2 * beta_ref[0] + xij
    z = jnp.maximum(
        jnp.dot(z, wl1_ref[...], preferred_element_type=jnp.float32)
        + bl1_ref[...], 0.0)
    out_ref[...] = jnp.dot(z, wl2_ref[...], preferred_element_type=jnp.float32) \
        + bl2_ref[...]


def _sc_gather(abits, x, ti, tj):
    b = ti.shape[0]
    info = plsc.get_sparse_core_info()
    nc, ns = info.num_cores, info.num_subcores
    per_w = b // (nc * ns)
    mesh = plsc.VectorSubcoreMesh(core_axis_name="c", subcore_axis_name="s")

    @functools.partial(
        pl.kernel, mesh=mesh,
        out_type=[
            jax.ShapeDtypeStruct((b, _WORDS), jnp.int32),
            jax.ShapeDtypeStruct((b, _WORDS), jnp.int32),
            jax.ShapeDtypeStruct((b, 128), jnp.float32),
            jax.ShapeDtypeStruct((b, 128), jnp.float32),
        ],
        scratch_types=[
            pltpu.VMEM((per_w,), jnp.int32),
            pltpu.VMEM((per_w, _WORDS), jnp.int32),
            pltpu.VMEM((per_w, 128), jnp.float32),
            pltpu.SemaphoreType.DMA,
        ],
    )
    def k(abits_hbm, x_hbm, ti_hbm, tj_hbm,
          r0_hbm, r1_hbm, xi_hbm, xj_hbm,
          idx_v, rows_v, xrows_v, sem):
        wid = lax.axis_index("s") * nc + lax.axis_index("c")
        base = wid * per_w
        pltpu.sync_copy(ti_hbm.at[pl.ds(base, per_w)], idx_v)
        pltpu.async_copy(abits_hbm.at[idx_v], rows_v, sem).wait()
        pltpu.sync_copy(rows_v, r0_hbm.at[pl.ds(base, per_w)])
        pltpu.async_copy(x_hbm.at[idx_v], xrows_v, sem).wait()
        pltpu.sync_copy(xrows_v, xi_hbm.at[pl.ds(base, per_w)])
        pltpu.sync_copy(tj_hbm.at[pl.ds(base, per_w)], idx_v)
        pltpu.async_copy(abits_hbm.at[idx_v], rows_v, sem).wait()
        pltpu.sync_copy(rows_v, r1_hbm.at[pl.ds(base, per_w)])
        pltpu.async_copy(x_hbm.at[idx_v], xrows_v, sem).wait()
        pltpu.sync_copy(xrows_v, xj_hbm.at[pl.ds(base, per_w)])

    return k(abits, x, ti, tj)


@jax.jit
def kernel(x, edge_index, tar_ei, beta,
           W_xlin1, b_xlin1, W_xlin2, b_xlin2,
           W_cn1, b_cn1, W_cn2, b_cn2,
           W_ij1, b_ij1, W_l1, b_l1, W_l2, b_l2):
    n, d = x.shape
    b = tar_ei.shape[1]

    src = edge_index[0].astype(jnp.int32)
    dst = edge_index[1].astype(jnp.int32)
    keys = (src << 14) | dst
    skeys, = lax.sort([keys], is_stable=False)
    dup = jnp.concatenate(
        [jnp.zeros((1,), jnp.bool_), skeys[1:] == skeys[:-1]])
    su = skeys >> 14
    sv = skeys & 16383
    word_idx = su * _WORDS + (sv >> 5)
    bitval = jnp.where(dup, 0, jnp.left_shift(jnp.int32(1), sv & 31))
    abits = jnp.zeros((n * _WORDS,), jnp.int32).at[word_idx].add(
        bitval, mode="drop").reshape(n, _WORDS)

    ti = tar_ei[0].astype(jnp.int32)
    tj = tar_ei[1].astype(jnp.int32)
    r0, r1, xi, xj = _sc_gather(abits, x, ti, tj)

    x_pad = jnp.pad(x, ((0, _N_PAD - n), (0, 0)))
    h3 = pl.pallas_call(
        _h_body,
        grid=(_N_PAD // _ROW_BLK,),
        in_specs=[
            pl.BlockSpec((_ROW_BLK, 128), lambda i: (i, 0)),
            pl.BlockSpec((128, 128), lambda i: (0, 0)),
            pl.BlockSpec((128,), lambda i: (0,)),
            pl.BlockSpec((128, 128), lambda i: (0, 0)),
            pl.BlockSpec((128,), lambda i: (0,)),
        ],
        out_specs=pl.BlockSpec((_ROW_BLK // 32, 32, 128), lambda i: (i, 0, 0)),
        out_shape=jax.ShapeDtypeStruct((_WORDS, 32, 128), jnp.float32),
    )(x_pad, W_xlin1, b_xlin1, W_xlin2, b_xlin2)
    h3 = jnp.pad(h3[: _N_PAD // 32], ((0, _WORDS - _N_PAD // 32), (0, 0), (0, 0)))

    beta_s = beta.astype(jnp.float32).reshape((1,))

    out = pl.pallas_call(
        _main_body,
        grid=(b // _TGT_BLK,),
        in_specs=[
            pl.BlockSpec((_TGT_BLK, _WORDS), lambda i: (i, 0)),
            pl.BlockSpec((_TGT_BLK, _WORDS), lambda i: (i, 0)),
            pl.BlockSpec((_TGT_BLK, 128), lambda i: (i, 0)),
            pl.BlockSpec((_TGT_BLK, 128), lambda i: (i, 0)),
            pl.BlockSpec((_WORDS, 32, 128), lambda i: (0, 0, 0)),
            pl.BlockSpec((128, 128), lambda i: (0, 0)),
            pl.BlockSpec((128,), lambda i: (0,)),
            pl.BlockSpec((128, 128), lambda i: (0, 0)),
            pl.BlockSpec((128,), lambda i: (0,)),
            pl.BlockSpec((128, 128), lambda i: (0, 0)),
            pl.BlockSpec((128,), lambda i: (0,)),
            pl.BlockSpec((128, 128), lambda i: (0, 0)),
            pl.BlockSpec((128,), lambda i: (0,)),
            pl.BlockSpec((128, 1), lambda i: (0, 0)),
            pl.BlockSpec((1,), lambda i: (0,)),
            pl.BlockSpec(memory_space=pltpu.SMEM),
        ],
        out_specs=pl.BlockSpec((_TGT_BLK, 1), lambda i: (i, 0)),
        out_shape=jax.ShapeDtypeStruct((b, 1), jnp.float32),
    )(r0, r1, xi, xj, h3,
      W_cn1, b_cn1, W_cn2, b_cn2,
      W_ij1, b_ij1, W_l1, b_l1, W_l2, b_l2,
      beta_s)
    return out
